# baseline (speedup 1.0000x reference)
import functools

import jax
import jax.numpy as jnp
from jax import lax
from jax.experimental import pallas as pl
from jax.experimental.pallas import tpu as pltpu
from jax.experimental.pallas import tpu_sc as plsc

EN = 16
KN = 2
DM = 768
DF = 3072
TN = 2048
RB = 256
RP = TN * KN + EN * RB
NBLK = RP // RB

NC, NS = 2, 16
NW = NC * NS
TPW = TN // NW
VL = 16


def _gelu_exact(v):
    return 0.5 * v * (1.0 + jax.lax.erf(v * 0.7071067811865476))


FCH = 4
FB = DF // FCH


def _gemm_body(base_ref, nblk_ref, x_ref, w0_ref, w1_ref, y_ref):
    e = pl.program_id(0)
    fi = pl.program_id(1)
    nb = nblk_ref[e]
    base = base_ref[e]
    w0b = w0_ref[0].astype(jnp.bfloat16)
    w1b = w1_ref[0].astype(jnp.bfloat16)

    def body(i, carry):
        sl = pl.ds(pl.multiple_of(base + i * RB, RB), RB)
        xb = x_ref[sl, :]
        h = _gelu_exact(jnp.dot(xb, w0b, preferred_element_type=jnp.float32))
        v = jnp.dot(h.astype(jnp.bfloat16), w1b, preferred_element_type=jnp.float32)

        @pl.when(fi == 0)
        def _():
            y_ref[sl, :] = v

        @pl.when(fi != 0)
        def _():
            y_ref[sl, :] = y_ref[sl, :] + v

        return carry

    lax.fori_loop(0, nb, body, 0)


def _grouped_gemm(base_rows, nblk, rep_x, w0, w1):
    return pl.pallas_call(
        _gemm_body,
        grid_spec=pltpu.PrefetchScalarGridSpec(
            num_scalar_prefetch=2,
            grid=(EN, FCH),
            in_specs=[
                pl.BlockSpec((RP, DM), lambda e, f, b, n: (0, 0)),
                pl.BlockSpec((1, DM, FB), lambda e, f, b, n: (e, 0, f)),
                pl.BlockSpec((1, FB, DM), lambda e, f, b, n: (e, f, 0)),
            ],
            out_specs=pl.BlockSpec((RP, DM), lambda e, f, b, n: (0, 0)),
        ),
        out_shape=jax.ShapeDtypeStruct((RP, DM), jnp.float32),
    )(base_rows, nblk, rep_x, w0, w1)


@functools.lru_cache(maxsize=1)
def _sc_mesh():
    return plsc.VectorSubcoreMesh(
        core_axis_name="c", subcore_axis_name="s", num_cores=NC, num_subcores=NS
    )


DM2 = DM // 2


def _sc_dispatch(x_bf, pos_e, pos_o):
    xi = jax.lax.bitcast_convert_type(x_bf.reshape(TN, DM2, 2), jnp.int32)
    ri = _sc_dispatch_kernel()(xi, pos_e, pos_o)
    return jax.lax.bitcast_convert_type(ri, jnp.bfloat16).reshape(RP, DM)


@functools.lru_cache(maxsize=1)
def _sc_dispatch_kernel():
    return functools.partial(
        pl.kernel,
        out_type=jax.ShapeDtypeStruct((RP, DM2), jnp.int32),
        mesh=_sc_mesh(),
        scratch_types=[
            pltpu.VMEM((TPW, DM2), jnp.int32),
            pltpu.VMEM((TPW,), jnp.int32),
            pltpu.VMEM((TPW,), jnp.int32),
            pltpu.SemaphoreType.DMA,
            pltpu.SemaphoreType.DMA,
        ],
    )(_sc_dispatch_body)


def _sc_dispatch_body(x_hbm, pe_hbm, po_hbm, repx_hbm, xbuf, pe_v, po_v, sem0, sem1):
    wid = lax.axis_index("s") * NC + lax.axis_index("c")
    base = wid * TPW
    pltpu.sync_copy(x_hbm.at[pl.ds(base, TPW)], xbuf)
    pltpu.sync_copy(pe_hbm.at[pl.ds(base, TPW)], pe_v)
    pltpu.sync_copy(po_hbm.at[pl.ds(base, TPW)], po_v)
    c0 = pltpu.async_copy(xbuf, repx_hbm.at[pe_v], sem0)
    c1 = pltpu.async_copy(xbuf, repx_hbm.at[po_v], sem1)
    c0.wait()
    c1.wait()


def _sc_combine(y, pos_e, pos_o):
    return _sc_combine_kernel()(y, pos_e, pos_o)


@functools.lru_cache(maxsize=1)
def _sc_combine_kernel():
    return functools.partial(
        pl.kernel,
        out_type=jax.ShapeDtypeStruct((TN, DM), jnp.float32),
        mesh=_sc_mesh(),
        scratch_types=[
            pltpu.VMEM((TPW, DM), jnp.float32),
            pltpu.VMEM((TPW, DM), jnp.float32),
            pltpu.VMEM((TPW,), jnp.int32),
            pltpu.VMEM((TPW,), jnp.int32),
            pltpu.SemaphoreType.DMA,
            pltpu.SemaphoreType.DMA,
        ],
    )(_sc_combine_body)


def _sc_combine_body(y_hbm, pe_hbm, po_hbm, out_hbm, ge, go, pe_v, po_v, sem0, sem1):
    wid = lax.axis_index("s") * NC + lax.axis_index("c")
    base = wid * TPW
    pltpu.sync_copy(pe_hbm.at[pl.ds(base, TPW)], pe_v)
    pltpu.sync_copy(po_hbm.at[pl.ds(base, TPW)], po_v)
    c0 = pltpu.async_copy(y_hbm.at[pe_v], ge, sem0)
    c1 = pltpu.async_copy(y_hbm.at[po_v], go, sem1)
    c0.wait()
    c1.wait()

    def row_add(r, carry):
        for s in range(DM // VL):
            sl = pl.ds(s * VL, VL)
            ge[r, sl] = ge[r, sl] + go[r, sl]
        return carry

    lax.fori_loop(0, TPW, row_add, 0)
    pltpu.sync_copy(ge, out_hbm.at[pl.ds(base, TPW)])


def kernel(x, topk_index, w0, w1):
    e = topk_index.reshape(-1)
    oh = (e[:, None] == jnp.arange(EN, dtype=e.dtype)).astype(jnp.int32)
    cs = jnp.cumsum(oh, axis=0)
    rank = jnp.sum((cs - oh) * oh, axis=1)
    counts = cs[-1]
    padded = ((counts + RB - 1) // RB) * RB
    base = jnp.concatenate(
        [jnp.zeros((1,), jnp.int32), jnp.cumsum(padded)[:-1].astype(jnp.int32)]
    )
    pos = rank + jnp.sum(oh * base[None, :], axis=1)
    nblk = (padded // RB).astype(jnp.int32)
    pos2 = pos.reshape(TN, KN)
    pos_e = pos2[:, 0]
    pos_o = pos2[:, 1]
    rep_x = _sc_dispatch(x.astype(jnp.bfloat16), pos_e, pos_o)
    y = _grouped_gemm(base, nblk, rep_x, w0, w1)
    return _sc_combine(y, pos_e, pos_o)


# speedup vs baseline: 1.0360x; 1.0360x over previous
import functools

import jax
import jax.numpy as jnp
from jax import lax
from jax.experimental import pallas as pl
from jax.experimental.pallas import tpu as pltpu
from jax.experimental.pallas import tpu_sc as plsc

EN = 16
KN = 2
DM = 768
DF = 3072
TN = 2048
RB = 256
RP = TN * KN + EN * RB
NBLK = RP // RB

NC, NS = 2, 16
NW = NC * NS
TPW = TN // NW
VL = 16


def _gelu_exact(v):
    return 0.5 * v * (1.0 + jax.lax.erf(v * 0.7071067811865476))


FCH = 3
FB = DF // FCH
SUBS = EN * FCH


def _issue_panels(step, slot, w0_hbm, w1_hbm, w0b, w1b, sem0, sem1):
    ee = step // FCH
    ff = pl.multiple_of((step % FCH) * FB, FB)
    pltpu.make_async_copy(
        w0_hbm.at[ee, :, pl.ds(ff, FB)], w0b.at[slot], sem0.at[slot]
    ).start()
    pltpu.make_async_copy(
        w1_hbm.at[ee, pl.ds(ff, FB), :], w1b.at[slot], sem1.at[slot]
    ).start()


def _gemm_body(base_ref, nblk_ref, x_ref, w0_hbm, w1_hbm, y_ref,
               w0b, w1b, sem0, sem1):
    s = pl.program_id(0)
    e = s // FCH
    fi = s % FCH
    slot = s % 2

    @pl.when(s == 0)
    def _():
        _issue_panels(s, slot, w0_hbm, w1_hbm, w0b, w1b, sem0, sem1)

    @pl.when(s + 1 < SUBS)
    def _():
        _issue_panels(s + 1, (s + 1) % 2, w0_hbm, w1_hbm, w0b, w1b, sem0, sem1)

    ee = e
    ff = pl.multiple_of(fi * FB, FB)
    pltpu.make_async_copy(
        w0_hbm.at[ee, :, pl.ds(ff, FB)], w0b.at[slot], sem0.at[slot]
    ).wait()
    pltpu.make_async_copy(
        w1_hbm.at[ee, pl.ds(ff, FB), :], w1b.at[slot], sem1.at[slot]
    ).wait()

    w0p = w0b[slot].astype(jnp.bfloat16)
    w1p = w1b[slot].astype(jnp.bfloat16)
    nb = nblk_ref[e]
    base = base_ref[e]

    def body(i, carry):
        sl = pl.ds(pl.multiple_of(base + i * RB, RB), RB)
        xb = x_ref[sl, :]
        h = _gelu_exact(jnp.dot(xb, w0p, preferred_element_type=jnp.float32))
        v = jnp.dot(h.astype(jnp.bfloat16), w1p, preferred_element_type=jnp.float32)

        @pl.when(fi == 0)
        def _():
            y_ref[sl, :] = v

        @pl.when(fi != 0)
        def _():
            y_ref[sl, :] = y_ref[sl, :] + v

        return carry

    lax.fori_loop(0, nb, body, 0)


def _grouped_gemm(base_rows, nblk, rep_x, w0, w1, interpret=False):
    return pl.pallas_call(
        _gemm_body,
        grid_spec=pltpu.PrefetchScalarGridSpec(
            num_scalar_prefetch=2,
            grid=(SUBS,),
            in_specs=[
                pl.BlockSpec((RP, DM), lambda s, b, n: (0, 0)),
                pl.BlockSpec(memory_space=pl.ANY),
                pl.BlockSpec(memory_space=pl.ANY),
            ],
            out_specs=pl.BlockSpec((RP, DM), lambda s, b, n: (0, 0)),
            scratch_shapes=[
                pltpu.VMEM((2, DM, FB), jnp.float32),
                pltpu.VMEM((2, FB, DM), jnp.float32),
                pltpu.SemaphoreType.DMA((2,)),
                pltpu.SemaphoreType.DMA((2,)),
            ],
        ),
        out_shape=jax.ShapeDtypeStruct((RP, DM), jnp.float32),
        interpret=interpret,
    )(base_rows, nblk, rep_x, w0, w1)


@functools.lru_cache(maxsize=1)
def _sc_mesh():
    return plsc.VectorSubcoreMesh(
        core_axis_name="c", subcore_axis_name="s", num_cores=NC, num_subcores=NS
    )


DM2 = DM // 2


def _sc_dispatch(x_bf, pos_e, pos_o):
    xi = jax.lax.bitcast_convert_type(x_bf.reshape(TN, DM2, 2), jnp.int32)
    ri = _sc_dispatch_kernel()(xi, pos_e, pos_o)
    return jax.lax.bitcast_convert_type(ri, jnp.bfloat16).reshape(RP, DM)


@functools.lru_cache(maxsize=1)
def _sc_dispatch_kernel():
    return functools.partial(
        pl.kernel,
        out_type=jax.ShapeDtypeStruct((RP, DM2), jnp.int32),
        mesh=_sc_mesh(),
        scratch_types=[
            pltpu.VMEM((TPW, DM2), jnp.int32),
            pltpu.VMEM((TPW,), jnp.int32),
            pltpu.VMEM((TPW,), jnp.int32),
            pltpu.SemaphoreType.DMA,
            pltpu.SemaphoreType.DMA,
        ],
    )(_sc_dispatch_body)


def _sc_dispatch_body(x_hbm, pe_hbm, po_hbm, repx_hbm, xbuf, pe_v, po_v, sem0, sem1):
    wid = lax.axis_index("s") * NC + lax.axis_index("c")
    base = wid * TPW
    pltpu.sync_copy(x_hbm.at[pl.ds(base, TPW)], xbuf)
    pltpu.sync_copy(pe_hbm.at[pl.ds(base, TPW)], pe_v)
    pltpu.sync_copy(po_hbm.at[pl.ds(base, TPW)], po_v)
    c0 = pltpu.async_copy(xbuf, repx_hbm.at[pe_v], sem0)
    c1 = pltpu.async_copy(xbuf, repx_hbm.at[po_v], sem1)
    c0.wait()
    c1.wait()


def _sc_combine(y, pos_e, pos_o):
    return _sc_combine_kernel()(y, pos_e, pos_o)


@functools.lru_cache(maxsize=1)
def _sc_combine_kernel():
    return functools.partial(
        pl.kernel,
        out_type=jax.ShapeDtypeStruct((TN, DM), jnp.float32),
        mesh=_sc_mesh(),
        scratch_types=[
            pltpu.VMEM((TPW, DM), jnp.float32),
            pltpu.VMEM((TPW, DM), jnp.float32),
            pltpu.VMEM((TPW,), jnp.int32),
            pltpu.VMEM((TPW,), jnp.int32),
            pltpu.SemaphoreType.DMA,
            pltpu.SemaphoreType.DMA,
        ],
    )(_sc_combine_body)


def _sc_combine_body(y_hbm, pe_hbm, po_hbm, out_hbm, ge, go, pe_v, po_v, sem0, sem1):
    wid = lax.axis_index("s") * NC + lax.axis_index("c")
    base = wid * TPW
    pltpu.sync_copy(pe_hbm.at[pl.ds(base, TPW)], pe_v)
    pltpu.sync_copy(po_hbm.at[pl.ds(base, TPW)], po_v)
    c0 = pltpu.async_copy(y_hbm.at[pe_v], ge, sem0)
    c1 = pltpu.async_copy(y_hbm.at[po_v], go, sem1)
    c0.wait()
    c1.wait()

    def row_add(r, carry):
        for s in range(DM // VL):
            sl = pl.ds(s * VL, VL)
            ge[r, sl] = ge[r, sl] + go[r, sl]
        return carry

    lax.fori_loop(0, TPW, row_add, 0)
    pltpu.sync_copy(ge, out_hbm.at[pl.ds(base, TPW)])


def kernel(x, topk_index, w0, w1):
    e = topk_index.reshape(-1)
    oh = (e[:, None] == jnp.arange(EN, dtype=e.dtype)).astype(jnp.int32)
    cs = jnp.cumsum(oh, axis=0)
    rank = jnp.sum((cs - oh) * oh, axis=1)
    counts = cs[-1]
    padded = ((counts + RB - 1) // RB) * RB
    base = jnp.concatenate(
        [jnp.zeros((1,), jnp.int32), jnp.cumsum(padded)[:-1].astype(jnp.int32)]
    )
    pos = rank + jnp.sum(oh * base[None, :], axis=1)
    nblk = (padded // RB).astype(jnp.int32)
    pos2 = pos.reshape(TN, KN)
    pos_e = pos2[:, 0]
    pos_o = pos2[:, 1]
    rep_x = _sc_dispatch(x.astype(jnp.bfloat16), pos_e, pos_o)
    y = _grouped_gemm(base, nblk, rep_x, w0, w1)
    return _sc_combine(y, pos_e, pos_o)
